# baseline (speedup 1.0000x reference)
import functools

import jax
import jax.numpy as jnp
import numpy as np
from jax import lax
from jax.experimental import pallas as pl
from jax.experimental.pallas import tpu as pltpu
from jax.experimental.pallas import tpu_sc as plsc

N = 10112
G = 79
A = 128
E = 161792
C = 16
L = 16


_EB = 2048


_NB = 128
_PINV = 1.0 / (128.0 + 1e-9)


def _front_body(ev_ref, ea_ref, x_ref, w0_ref, wp_ref,
                zt_ref, x0_ref, prt_ref, px_ref):
    x = ev_ref[0:1, :]
    y = ev_ref[1:2, :]
    z = ev_ref[2:3, :]
    inv = 1.0 / (jnp.sqrt(x * x + y * y + z * z) + 1e-9)
    x = x * inv
    y = y * inv
    z = z * inv
    s = ea_ref[0:1, :] * 0.1
    o = jnp.ones_like(x)
    xx = x * x
    yy = y * y
    zz = z * z
    cols = [
        o,
        x,
        y,
        z,
        1.7320508 * x * y,
        1.7320508 * y * z,
        0.5 * (3.0 * zz - 1.0),
        1.7320508 * x * z,
        0.8660254 * (xx - yy),
        y * (3.0 * xx - yy),
        x * y * z,
        y * (5.0 * zz - 1.0),
        z * (5.0 * zz - 3.0),
        x * (5.0 * zz - 1.0),
        z * (xx - yy),
        x * (xx - 3.0 * yy),
    ]
    zt_ref[...] = jnp.concatenate([c * s for c in cols], axis=0)

    x0 = jnp.tanh(jnp.dot(x_ref[...], w0_ref[...],
                          preferred_element_type=jnp.float32))
    x0_ref[...] = x0
    prt_ref[...] = lax.dot_general(wp_ref[...], x0, (((0,), (1,)), ((), ())),
                                   preferred_element_type=jnp.float32)
    px_ref[...] = (jnp.sum(x0, 0, keepdims=True) * _PINV).reshape(1, 1, 64)


def _front_stage(edge_vectors, edge_attr_t, x, w0, wproj0):
    return pl.pallas_call(
        _front_body,
        grid=(G,),
        in_specs=[
            pl.BlockSpec((3, _EB), lambda i: (0, i)),
            pl.BlockSpec((1, _EB), lambda i: (0, i)),
            pl.BlockSpec((_NB, A), lambda i: (i, 0)),
            pl.BlockSpec((A, 64), lambda i: (0, 0)),
            pl.BlockSpec((64, C), lambda i: (0, 0)),
        ],
        out_specs=[
            pl.BlockSpec((L, _EB), lambda i: (0, i)),
            pl.BlockSpec((_NB, 64), lambda i: (i, 0)),
            pl.BlockSpec((C, _NB), lambda i: (0, i)),
            pl.BlockSpec((1, 1, 64), lambda i: (i, 0, 0)),
        ],
        out_shape=[
            jax.ShapeDtypeStruct((L, E), jnp.float32),
            jax.ShapeDtypeStruct((N, 64), jnp.float32),
            jax.ShapeDtypeStruct((C, N), jnp.float32),
            jax.ShapeDtypeStruct((G, 1, 64), jnp.float32),
        ],
    )(edge_vectors, edge_attr_t, x, w0, wproj0)


_SCB = 1024
_NBLK = E // _SCB
_CT = 4
_LT = 2

_sc_mesh = plsc.VectorSubcoreMesh(core_axis_name="c", subcore_axis_name="s")
_TRI = A * (A - 1) // 2

_SC_SCRATCH = [
    pltpu.VMEM((_CT, N), jnp.float32),
    pltpu.VMEM((_CT * _LT * N,), jnp.float32),
    pltpu.VMEM((2, _SCB), jnp.int32),
    pltpu.VMEM((2, _SCB), jnp.int32),
    pltpu.VMEM((2, _LT, _SCB), jnp.float32),
    pltpu.SemaphoreType.DMA((2,)),
    pltpu.SemaphoreType.DMA((2,)),
    pltpu.SemaphoreType.DMA((2,)),
    pltpu.SemaphoreType.DMA,
]


def _sc_triu_phase(xg_hbm, tidxf_hbm, xt_hbm, acc, psem, ci, si):
    w = si * 2 + ci
    pltpu.async_copy(tidxf_hbm, acc.at[pl.ds(0, _TRI)], psem).wait()
    for rep in range(3):
        g = w + rep * 32

        @pl.when(g < G)
        def _():
            pltpu.async_copy(xg_hbm.at[g], acc.at[pl.ds(8192, A * A)],
                             psem).wait()

            @plsc.parallel_loop(0, _TRI // 16, 1, unroll=4)
            def _grp(q):
                ii = plsc.bitcast(acc[pl.ds(q * 16, 16)], jnp.int32)
                acc[pl.ds(24576 + q * 16, 16)] = plsc.load_gather(acc, [ii])

            pltpu.sync_copy(acc.at[pl.ds(24576, _TRI)],
                            xt_hbm.at[pl.ds(g * _TRI, _TRI)])


def _sc_body(prt_hbm, zt_hbm, src_hbm, dst_hbm, out_hbm,
             prc, acc, srcb, dstb, zb, ssem, dsem, zsem, psem):
    ci = lax.axis_index("c")
    si = lax.axis_index("s")
    cq = lax.rem(si, 4)
    lp = lax.div(si, 4) + 4 * ci
    lbase = lp * _LT

    pltpu.async_copy(prt_hbm.at[pl.ds(cq * _CT, _CT)], prc, psem).wait()

    zeros = jnp.zeros((16,), jnp.float32)

    @plsc.parallel_loop(0, _CT * _LT * N // 16, 1, unroll=8)
    def _zero(i):
        acc[pl.ds(i * 16, 16)] = zeros

    def _issue(g, slot):
        base = g * _SCB
        cs = pltpu.async_copy(src_hbm.at[pl.ds(base, _SCB)], srcb.at[slot],
                              ssem.at[slot])
        cd = pltpu.async_copy(dst_hbm.at[pl.ds(base, _SCB)], dstb.at[slot],
                              dsem.at[slot])
        cz = pltpu.async_copy(
            zt_hbm.at[pl.ds(lbase, _LT), pl.ds(base, _SCB)], zb.at[slot],
            zsem.at[slot])
        del cs, cd, cz

    def _wait(slot):
        pltpu.make_async_copy(src_hbm.at[pl.ds(0, _SCB)], srcb.at[slot],
                              ssem.at[slot]).wait()
        pltpu.make_async_copy(dst_hbm.at[pl.ds(0, _SCB)], dstb.at[slot],
                              dsem.at[slot]).wait()
        pltpu.make_async_copy(
            zt_hbm.at[pl.ds(0, _LT), pl.ds(0, _SCB)], zb.at[slot],
            zsem.at[slot]).wait()

    _issue(0, 0)

    def _block(g, _):
        slot = lax.rem(g, 2)
        _wait(slot)

        @pl.when(g + 1 < _NBLK)
        def _():
            _issue(g + 1, 1 - slot)

        @plsc.parallel_loop(0, _SCB // 16, 1, unroll=4)
        def _group(q):
            off = q * 16
            s = srcb[slot, pl.ds(off, 16)]
            d = dstb[slot, pl.ds(off, 16)]
            z0 = zb[slot, 0, pl.ds(off, 16)]
            z1 = zb[slot, 1, pl.ds(off, 16)]
            for cl in range(_CT):
                u = plsc.load_gather(
                    prc, [jnp.full((16,), cl, jnp.int32), s])
                plsc.addupdate_scatter(acc, [d + (2 * cl) * N], u * z0)
                plsc.addupdate_scatter(acc, [d + (2 * cl + 1) * N], u * z1)
        return 0

    lax.fori_loop(0, _NBLK, _block, 0)

    for cl in range(_CT):
        for ll in range(_LT):
            j = (cq * _CT + cl) * L + lbase + ll
            pltpu.sync_copy(acc.at[pl.ds((cl * _LT + ll) * N, N)],
                            out_hbm.at[pl.ds(j * N, N)])


@functools.partial(
    pl.kernel,
    out_type=jax.ShapeDtypeStruct((C * L * N,), jnp.float32),
    mesh=_sc_mesh,
    scratch_types=list(_SC_SCRATCH),
    compiler_params=pltpu.CompilerParams(needs_layout_passes=False),
)
def _sc_scatter(prt_hbm, zt_hbm, src_hbm, dst_hbm, out_hbm,
                prc, acc, srcb, dstb, zb, ssem, dsem, zsem, psem):
    _sc_body(prt_hbm, zt_hbm, src_hbm, dst_hbm, out_hbm,
             prc, acc, srcb, dstb, zb, ssem, dsem, zsem, psem)


@functools.partial(
    pl.kernel,
    out_type=[
        jax.ShapeDtypeStruct((C * L * N,), jnp.float32),
        jax.ShapeDtypeStruct((G * _TRI,), jnp.float32),
    ],
    mesh=_sc_mesh,
    scratch_types=list(_SC_SCRATCH),
    compiler_params=pltpu.CompilerParams(needs_layout_passes=False),
)
def _sc_scatter_triu(prt_hbm, zt_hbm, src_hbm, dst_hbm, xg_hbm, tidxf_hbm,
                     out_hbm, xt_hbm,
                     prc, acc, srcb, dstb, zb, ssem, dsem, zsem, psem):
    ci = lax.axis_index("c")
    si = lax.axis_index("s")
    _sc_triu_phase(xg_hbm, tidxf_hbm, xt_hbm, acc, psem, ci, si)
    _sc_body(prt_hbm, zt_hbm, src_hbm, dst_hbm, out_hbm,
             prc, acc, srcb, dstb, zb, ssem, dsem, zsem, psem)


def _dense_body_pr(aft_ref, nf_ref, x_ref, w1_ref, w2_ref, w3_ref,
                   wsc_ref, wat_ref, wpn_ref, h_ref, prt_ref, ph_ref):
    aft = aft_ref[...]
    dn = (((0,), (0,)), ((), ()))
    b = lax.dot_general(aft, w1_ref[...], dn,
                        preferred_element_type=jnp.float32)
    af2 = aft * aft
    b += lax.dot_general(af2, w2_ref[...], dn,
                         preferred_element_type=jnp.float32)
    b += lax.dot_general(af2 * aft, w3_ref[...], dn,
                         preferred_element_type=jnp.float32)
    sc = (jnp.dot(nf_ref[...], wsc_ref[...],
                  preferred_element_type=jnp.float32)
          * jnp.dot(x_ref[...], wat_ref[...],
                    preferred_element_type=jnp.float32))
    h = jnp.tanh(b + sc)
    h_ref[...] = h
    prt_ref[...] = lax.dot_general(wpn_ref[...], h, (((0,), (1,)), ((), ())),
                                   preferred_element_type=jnp.float32)
    do = h.shape[1]
    ph_ref[...] = (jnp.sum(h, 0, keepdims=True) * _PINV).reshape(1, 1, do)


def _dense_body(aft_ref, nf_ref, x_ref, w1_ref, w2_ref, w3_ref,
                wsc_ref, wat_ref, h_ref, ph_ref):
    aft = aft_ref[...]
    dn = (((0,), (0,)), ((), ()))
    b = lax.dot_general(aft, w1_ref[...], dn,
                        preferred_element_type=jnp.float32)
    af2 = aft * aft
    b += lax.dot_general(af2, w2_ref[...], dn,
                         preferred_element_type=jnp.float32)
    b += lax.dot_general(af2 * aft, w3_ref[...], dn,
                         preferred_element_type=jnp.float32)
    sc = (jnp.dot(nf_ref[...], wsc_ref[...],
                  preferred_element_type=jnp.float32)
          * jnp.dot(x_ref[...], wat_ref[...],
                    preferred_element_type=jnp.float32))
    h = jnp.tanh(b + sc)
    h_ref[...] = h
    do = h.shape[1]
    ph_ref[...] = (jnp.sum(h, 0, keepdims=True) * _PINV).reshape(1, 1, do)


def _dense_stage(aft, nf, x, p, wproj_next):
    di = nf.shape[1]
    do = p["W1"].shape[1]
    in_specs = [
        pl.BlockSpec((C * L, _NB), lambda i: (0, i)),
        pl.BlockSpec((_NB, di), lambda i: (i, 0)),
        pl.BlockSpec((_NB, A), lambda i: (i, 0)),
        pl.BlockSpec((C * L, do), lambda i: (0, 0)),
        pl.BlockSpec((C * L, do), lambda i: (0, 0)),
        pl.BlockSpec((C * L, do), lambda i: (0, 0)),
        pl.BlockSpec((di, do), lambda i: (0, 0)),
        pl.BlockSpec((A, do), lambda i: (0, 0)),
    ]
    args = [aft, nf, x, p["W1"], p["W2"], p["W3"], p["Wsc"], p["Wattr"]]
    if wproj_next is not None:
        in_specs.append(pl.BlockSpec((do, C), lambda i: (0, 0)))
        args.append(wproj_next)
        return pl.pallas_call(
            _dense_body_pr,
            grid=(N // _NB,),
            in_specs=in_specs,
            out_specs=[
                pl.BlockSpec((_NB, do), lambda i: (i, 0)),
                pl.BlockSpec((C, _NB), lambda i: (0, i)),
                pl.BlockSpec((1, 1, do), lambda i: (i, 0, 0)),
            ],
            out_shape=[
                jax.ShapeDtypeStruct((N, do), jnp.float32),
                jax.ShapeDtypeStruct((C, N), jnp.float32),
                jax.ShapeDtypeStruct((G, 1, do), jnp.float32),
            ],
        )(*args)
    return pl.pallas_call(
        _dense_body,
        grid=(N // _NB,),
        in_specs=in_specs,
        out_specs=[
            pl.BlockSpec((_NB, do), lambda i: (i, 0)),
            pl.BlockSpec((1, 1, do), lambda i: (i, 0, 0)),
        ],
        out_shape=[
            jax.ShapeDtypeStruct((N, do), jnp.float32),
            jax.ShapeDtypeStruct((G, 1, do), jnp.float32),
        ],
    )(*args)


def _bn_val(z, g, b):
    m = jnp.mean(z, axis=0, keepdims=True)
    v = jnp.mean((z - m) * (z - m), axis=0, keepdims=True)
    return (z - m) * lax.rsqrt(v + 1e-5) * g + b


def _head_body(xt_ref, ph_ref, bng_ref, bnb_ref, bhg_ref, bhb_ref,
               w0a_ref, w0b_ref, b0_ref, g0_ref, bb0_ref,
               w1_ref, b1_ref, g1_ref, bb1_ref,
               w2_ref, b2_ref, g2_ref, bb2_ref,
               w3_ref, b3_ref, out_ref):
    xt = _bn_val(xt_ref[...], bng_ref[...], bnb_ref[...])
    h = _bn_val(ph_ref[...], bhg_ref[...], bhb_ref[...])
    z = (jnp.dot(xt, w0a_ref[...], preferred_element_type=jnp.float32)
         + jnp.dot(h, w0b_ref[...], preferred_element_type=jnp.float32)
         + b0_ref[...])
    z = jnp.maximum(_bn_val(z, g0_ref[...], bb0_ref[...]), 0.0)
    z = jnp.dot(z, w1_ref[...], preferred_element_type=jnp.float32) + b1_ref[...]
    z = jnp.maximum(_bn_val(z, g1_ref[...], bb1_ref[...]), 0.0)
    z = jnp.dot(z, w2_ref[...], preferred_element_type=jnp.float32) + b2_ref[...]
    z = jnp.maximum(_bn_val(z, g2_ref[...], bb2_ref[...]), 0.0)
    lg = jnp.dot(z, w3_ref[...], preferred_element_type=jnp.float32) + b3_ref[...]
    mx = jnp.max(lg, axis=1, keepdims=True)
    sh = lg - mx
    lse = jnp.log(jnp.sum(jnp.exp(sh), axis=1, keepdims=True))
    out_ref[...] = sh - lse


def _head(xt, ph, p):
    args = [
        xt, ph,
        p["bn_g"].reshape(1, _TRI), p["bn_b"].reshape(1, _TRI),
        p["bnh_g"].reshape(1, 640), p["bnh_b"].reshape(1, 640),
        p["mW0"][:_TRI], p["mW0"][_TRI:], p["mb0"].reshape(1, 512),
        p["mg0"].reshape(1, 512), p["mbb0"].reshape(1, 512),
        p["mW1"], p["mb1"].reshape(1, 256),
        p["mg1"].reshape(1, 256), p["mbb1"].reshape(1, 256),
        p["mW2"], p["mb2"].reshape(1, 256),
        p["mg2"].reshape(1, 256), p["mbb2"].reshape(1, 256),
        p["mW3"], p["mb3"].reshape(1, 10),
    ]
    return pl.pallas_call(
        _head_body,
        out_shape=jax.ShapeDtypeStruct((G, 10), jnp.float32),
    )(*args)


_IU, _JU = np.triu_indices(A, k=1)
_TIDX = np.asarray(_IU * A + _JU + 8192, dtype=np.int32)


def kernel(x, edge_vectors, edge_attr, edge_index, batch, params):
    del batch
    src = edge_index[0].astype(jnp.int32)
    dst = edge_index[1].astype(jnp.int32)
    ea_t = edge_attr.reshape(1, E)

    zt, x0, prt, px = _front_stage(edge_vectors, ea_t, x,
                                   params["W0"], params["l0"]["Wproj"])

    tidxf = jnp.asarray(_TIDX).view(jnp.float32)
    pools = [px.reshape(G, 64)]
    hs = []
    nf = x0
    xt = None
    for i in range(3):
        p = params["l%d" % i]
        if i == 0:
            aft, xt = _sc_scatter_triu(prt, zt, src, dst,
                                       x.reshape(G, A * A), tidxf)
            xt = xt.reshape(G, _TRI)
        else:
            aft = _sc_scatter(prt, zt, src, dst)
        aft = aft.reshape(C * L, N)
        wpn = params["l%d" % (i + 1)]["Wproj"] if i < 2 else None
        if wpn is not None:
            h, prt, ph = _dense_stage(aft, nf, x, p, wpn)
        else:
            h, ph = _dense_stage(aft, nf, x, p, None)
        pools.append(ph.reshape(G, p["W1"].shape[1]))
        hs.append(h)
        nf = h

    pooled = jnp.concatenate(pools, axis=1)
    return _head(xt, pooled, params)


# speedup vs baseline: 1.0187x; 1.0187x over previous
import functools

import jax
import jax.numpy as jnp
import numpy as np
from jax import lax
from jax.experimental import pallas as pl
from jax.experimental.pallas import tpu as pltpu
from jax.experimental.pallas import tpu_sc as plsc

N = 10112
G = 79
A = 128
E = 161792
C = 16
L = 16


_EB = 2048


_NB = 128
_PINV = 1.0 / (128.0 + 1e-9)


def _front_body(ev_ref, ea_ref, x_ref, w0_ref, wp_ref,
                zt_ref, x0_ref, prt_ref, px_ref):
    x = ev_ref[0:1, :]
    y = ev_ref[1:2, :]
    z = ev_ref[2:3, :]
    inv = 1.0 / (jnp.sqrt(x * x + y * y + z * z) + 1e-9)
    x = x * inv
    y = y * inv
    z = z * inv
    s = ea_ref[0:1, :] * 0.1
    o = jnp.ones_like(x)
    xx = x * x
    yy = y * y
    zz = z * z
    cols = [
        o,
        x,
        y,
        z,
        1.7320508 * x * y,
        1.7320508 * y * z,
        0.5 * (3.0 * zz - 1.0),
        1.7320508 * x * z,
        0.8660254 * (xx - yy),
        y * (3.0 * xx - yy),
        x * y * z,
        y * (5.0 * zz - 1.0),
        z * (5.0 * zz - 3.0),
        x * (5.0 * zz - 1.0),
        z * (xx - yy),
        x * (xx - 3.0 * yy),
    ]
    zt_ref[...] = jnp.concatenate([c * s for c in cols], axis=0)

    x0 = jnp.tanh(jnp.dot(x_ref[...], w0_ref[...],
                          preferred_element_type=jnp.float32))
    x0_ref[...] = x0
    prt_ref[...] = lax.dot_general(wp_ref[...], x0, (((0,), (1,)), ((), ())),
                                   preferred_element_type=jnp.float32)
    px_ref[...] = (jnp.sum(x0, 0, keepdims=True) * _PINV).reshape(1, 1, 64)


def _front_stage(edge_vectors, edge_attr_t, x, w0, wproj0):
    return pl.pallas_call(
        _front_body,
        grid=(G,),
        in_specs=[
            pl.BlockSpec((3, _EB), lambda i: (0, i)),
            pl.BlockSpec((1, _EB), lambda i: (0, i)),
            pl.BlockSpec((_NB, A), lambda i: (i, 0)),
            pl.BlockSpec((A, 64), lambda i: (0, 0)),
            pl.BlockSpec((64, C), lambda i: (0, 0)),
        ],
        out_specs=[
            pl.BlockSpec((L, _EB), lambda i: (0, i)),
            pl.BlockSpec((_NB, 64), lambda i: (i, 0)),
            pl.BlockSpec((C, _NB), lambda i: (0, i)),
            pl.BlockSpec((1, 1, 64), lambda i: (i, 0, 0)),
        ],
        out_shape=[
            jax.ShapeDtypeStruct((L, E), jnp.float32),
            jax.ShapeDtypeStruct((N, 64), jnp.float32),
            jax.ShapeDtypeStruct((C, N), jnp.float32),
            jax.ShapeDtypeStruct((G, 1, 64), jnp.float32),
        ],
    )(edge_vectors, edge_attr_t, x, w0, wproj0)


_SCB = 2048
_NBLK = E // _SCB
_CT = 2
_LT = 4

_sc_mesh = plsc.VectorSubcoreMesh(core_axis_name="c", subcore_axis_name="s")
_TRI = A * (A - 1) // 2

_SC_SCRATCH = [
    pltpu.VMEM((_CT, N), jnp.float32),
    pltpu.VMEM((_CT * _LT * N,), jnp.float32),
    pltpu.VMEM((2, _SCB), jnp.int32),
    pltpu.VMEM((2, _SCB), jnp.int32),
    pltpu.VMEM((2, _LT, _SCB), jnp.float32),
    pltpu.SemaphoreType.DMA((2,)),
    pltpu.SemaphoreType.DMA((2,)),
    pltpu.SemaphoreType.DMA((2,)),
    pltpu.SemaphoreType.DMA,
]


def _sc_triu_phase(xg_hbm, tidxf_hbm, xt_hbm, acc, psem, ci, si):
    w = si * 2 + ci
    pltpu.async_copy(tidxf_hbm, acc.at[pl.ds(0, _TRI)], psem).wait()
    for rep in range(3):
        g = w + rep * 32

        @pl.when(g < G)
        def _():
            pltpu.async_copy(xg_hbm.at[g], acc.at[pl.ds(8192, A * A)],
                             psem).wait()

            @plsc.parallel_loop(0, _TRI // 16, 1, unroll=4)
            def _grp(q):
                ii = plsc.bitcast(acc[pl.ds(q * 16, 16)], jnp.int32)
                acc[pl.ds(24576 + q * 16, 16)] = plsc.load_gather(acc, [ii])

            pltpu.sync_copy(acc.at[pl.ds(24576, _TRI)],
                            xt_hbm.at[pl.ds(g * _TRI, _TRI)])


def _sc_body(prt_hbm, zt_hbm, src_hbm, dst_hbm, out_hbm,
             prc, acc, srcb, dstb, zb, ssem, dsem, zsem, psem):
    ci = lax.axis_index("c")
    si = lax.axis_index("s")
    cq = lax.rem(si, L // _CT)
    lp = lax.div(si, L // _CT) + (L // (_CT * _LT)) * ci
    lbase = lp * _LT

    pltpu.async_copy(prt_hbm.at[pl.ds(cq * _CT, _CT)], prc, psem).wait()

    zeros = jnp.zeros((16,), jnp.float32)

    @plsc.parallel_loop(0, _CT * _LT * N // 16, 1, unroll=8)
    def _zero(i):
        acc[pl.ds(i * 16, 16)] = zeros

    def _issue(g, slot):
        base = g * _SCB
        cs = pltpu.async_copy(src_hbm.at[pl.ds(base, _SCB)], srcb.at[slot],
                              ssem.at[slot])
        cd = pltpu.async_copy(dst_hbm.at[pl.ds(base, _SCB)], dstb.at[slot],
                              dsem.at[slot])
        cz = pltpu.async_copy(
            zt_hbm.at[pl.ds(lbase, _LT), pl.ds(base, _SCB)], zb.at[slot],
            zsem.at[slot])
        del cs, cd, cz

    def _wait(slot):
        pltpu.make_async_copy(src_hbm.at[pl.ds(0, _SCB)], srcb.at[slot],
                              ssem.at[slot]).wait()
        pltpu.make_async_copy(dst_hbm.at[pl.ds(0, _SCB)], dstb.at[slot],
                              dsem.at[slot]).wait()
        pltpu.make_async_copy(
            zt_hbm.at[pl.ds(0, _LT), pl.ds(0, _SCB)], zb.at[slot],
            zsem.at[slot]).wait()

    _issue(0, 0)

    def _block(g, _):
        slot = lax.rem(g, 2)
        _wait(slot)

        @pl.when(g + 1 < _NBLK)
        def _():
            _issue(g + 1, 1 - slot)

        @plsc.parallel_loop(0, _SCB // 16, 1, unroll=4)
        def _group(q):
            off = q * 16
            s = srcb[slot, pl.ds(off, 16)]
            d = dstb[slot, pl.ds(off, 16)]
            zs = [zb[slot, ll, pl.ds(off, 16)] for ll in range(_LT)]
            for cl in range(_CT):
                u = plsc.load_gather(
                    prc, [jnp.full((16,), cl, jnp.int32), s])
                for ll in range(_LT):
                    plsc.addupdate_scatter(
                        acc, [d + (cl * _LT + ll) * N], u * zs[ll])
        return 0

    lax.fori_loop(0, _NBLK, _block, 0)

    for cl in range(_CT):
        for ll in range(_LT):
            j = (cq * _CT + cl) * L + lbase + ll
            pltpu.sync_copy(acc.at[pl.ds((cl * _LT + ll) * N, N)],
                            out_hbm.at[pl.ds(j * N, N)])


@functools.partial(
    pl.kernel,
    out_type=jax.ShapeDtypeStruct((C * L * N,), jnp.float32),
    mesh=_sc_mesh,
    scratch_types=list(_SC_SCRATCH),
    compiler_params=pltpu.CompilerParams(needs_layout_passes=False),
)
def _sc_scatter(prt_hbm, zt_hbm, src_hbm, dst_hbm, out_hbm,
                prc, acc, srcb, dstb, zb, ssem, dsem, zsem, psem):
    _sc_body(prt_hbm, zt_hbm, src_hbm, dst_hbm, out_hbm,
             prc, acc, srcb, dstb, zb, ssem, dsem, zsem, psem)


@functools.partial(
    pl.kernel,
    out_type=[
        jax.ShapeDtypeStruct((C * L * N,), jnp.float32),
        jax.ShapeDtypeStruct((G * _TRI,), jnp.float32),
    ],
    mesh=_sc_mesh,
    scratch_types=list(_SC_SCRATCH),
    compiler_params=pltpu.CompilerParams(needs_layout_passes=False),
)
def _sc_scatter_triu(prt_hbm, zt_hbm, src_hbm, dst_hbm, xg_hbm, tidxf_hbm,
                     out_hbm, xt_hbm,
                     prc, acc, srcb, dstb, zb, ssem, dsem, zsem, psem):
    ci = lax.axis_index("c")
    si = lax.axis_index("s")
    _sc_triu_phase(xg_hbm, tidxf_hbm, xt_hbm, acc, psem, ci, si)
    _sc_body(prt_hbm, zt_hbm, src_hbm, dst_hbm, out_hbm,
             prc, acc, srcb, dstb, zb, ssem, dsem, zsem, psem)


def _dense_body_pr(aft_ref, nf_ref, x_ref, w1_ref, w2_ref, w3_ref,
                   wsc_ref, wat_ref, wpn_ref, h_ref, prt_ref, ph_ref):
    aft = aft_ref[...]
    dn = (((0,), (0,)), ((), ()))
    b = lax.dot_general(aft, w1_ref[...], dn,
                        preferred_element_type=jnp.float32)
    af2 = aft * aft
    b += lax.dot_general(af2, w2_ref[...], dn,
                         preferred_element_type=jnp.float32)
    b += lax.dot_general(af2 * aft, w3_ref[...], dn,
                         preferred_element_type=jnp.float32)
    sc = (jnp.dot(nf_ref[...], wsc_ref[...],
                  preferred_element_type=jnp.float32)
          * jnp.dot(x_ref[...], wat_ref[...],
                    preferred_element_type=jnp.float32))
    h = jnp.tanh(b + sc)
    h_ref[...] = h
    prt_ref[...] = lax.dot_general(wpn_ref[...], h, (((0,), (1,)), ((), ())),
                                   preferred_element_type=jnp.float32)
    do = h.shape[1]
    ph_ref[...] = (jnp.sum(h, 0, keepdims=True) * _PINV).reshape(1, 1, do)


def _dense_body(aft_ref, nf_ref, x_ref, w1_ref, w2_ref, w3_ref,
                wsc_ref, wat_ref, h_ref, ph_ref):
    aft = aft_ref[...]
    dn = (((0,), (0,)), ((), ()))
    b = lax.dot_general(aft, w1_ref[...], dn,
                        preferred_element_type=jnp.float32)
    af2 = aft * aft
    b += lax.dot_general(af2, w2_ref[...], dn,
                         preferred_element_type=jnp.float32)
    b += lax.dot_general(af2 * aft, w3_ref[...], dn,
                         preferred_element_type=jnp.float32)
    sc = (jnp.dot(nf_ref[...], wsc_ref[...],
                  preferred_element_type=jnp.float32)
          * jnp.dot(x_ref[...], wat_ref[...],
                    preferred_element_type=jnp.float32))
    h = jnp.tanh(b + sc)
    h_ref[...] = h
    do = h.shape[1]
    ph_ref[...] = (jnp.sum(h, 0, keepdims=True) * _PINV).reshape(1, 1, do)


def _dense_stage(aft, nf, x, p, wproj_next):
    di = nf.shape[1]
    do = p["W1"].shape[1]
    in_specs = [
        pl.BlockSpec((C * L, _NB), lambda i: (0, i)),
        pl.BlockSpec((_NB, di), lambda i: (i, 0)),
        pl.BlockSpec((_NB, A), lambda i: (i, 0)),
        pl.BlockSpec((C * L, do), lambda i: (0, 0)),
        pl.BlockSpec((C * L, do), lambda i: (0, 0)),
        pl.BlockSpec((C * L, do), lambda i: (0, 0)),
        pl.BlockSpec((di, do), lambda i: (0, 0)),
        pl.BlockSpec((A, do), lambda i: (0, 0)),
    ]
    args = [aft, nf, x, p["W1"], p["W2"], p["W3"], p["Wsc"], p["Wattr"]]
    if wproj_next is not None:
        in_specs.append(pl.BlockSpec((do, C), lambda i: (0, 0)))
        args.append(wproj_next)
        return pl.pallas_call(
            _dense_body_pr,
            grid=(N // _NB,),
            in_specs=in_specs,
            out_specs=[
                pl.BlockSpec((_NB, do), lambda i: (i, 0)),
                pl.BlockSpec((C, _NB), lambda i: (0, i)),
                pl.BlockSpec((1, 1, do), lambda i: (i, 0, 0)),
            ],
            out_shape=[
                jax.ShapeDtypeStruct((N, do), jnp.float32),
                jax.ShapeDtypeStruct((C, N), jnp.float32),
                jax.ShapeDtypeStruct((G, 1, do), jnp.float32),
            ],
        )(*args)
    return pl.pallas_call(
        _dense_body,
        grid=(N // _NB,),
        in_specs=in_specs,
        out_specs=[
            pl.BlockSpec((_NB, do), lambda i: (i, 0)),
            pl.BlockSpec((1, 1, do), lambda i: (i, 0, 0)),
        ],
        out_shape=[
            jax.ShapeDtypeStruct((N, do), jnp.float32),
            jax.ShapeDtypeStruct((G, 1, do), jnp.float32),
        ],
    )(*args)


def _bn_val(z, g, b):
    m = jnp.mean(z, axis=0, keepdims=True)
    v = jnp.mean((z - m) * (z - m), axis=0, keepdims=True)
    return (z - m) * lax.rsqrt(v + 1e-5) * g + b


def _head_body(xt_ref, ph_ref, bng_ref, bnb_ref, bhg_ref, bhb_ref,
               w0a_ref, w0b_ref, b0_ref, g0_ref, bb0_ref,
               w1_ref, b1_ref, g1_ref, bb1_ref,
               w2_ref, b2_ref, g2_ref, bb2_ref,
               w3_ref, b3_ref, out_ref):
    xt = _bn_val(xt_ref[...], bng_ref[...], bnb_ref[...])
    h = _bn_val(ph_ref[...], bhg_ref[...], bhb_ref[...])
    z = (jnp.dot(xt, w0a_ref[...], preferred_element_type=jnp.float32)
         + jnp.dot(h, w0b_ref[...], preferred_element_type=jnp.float32)
         + b0_ref[...])
    z = jnp.maximum(_bn_val(z, g0_ref[...], bb0_ref[...]), 0.0)
    z = jnp.dot(z, w1_ref[...], preferred_element_type=jnp.float32) + b1_ref[...]
    z = jnp.maximum(_bn_val(z, g1_ref[...], bb1_ref[...]), 0.0)
    z = jnp.dot(z, w2_ref[...], preferred_element_type=jnp.float32) + b2_ref[...]
    z = jnp.maximum(_bn_val(z, g2_ref[...], bb2_ref[...]), 0.0)
    lg = jnp.dot(z, w3_ref[...], preferred_element_type=jnp.float32) + b3_ref[...]
    mx = jnp.max(lg, axis=1, keepdims=True)
    sh = lg - mx
    lse = jnp.log(jnp.sum(jnp.exp(sh), axis=1, keepdims=True))
    out_ref[...] = sh - lse


def _head(xt, ph, p):
    args = [
        xt, ph,
        p["bn_g"].reshape(1, _TRI), p["bn_b"].reshape(1, _TRI),
        p["bnh_g"].reshape(1, 640), p["bnh_b"].reshape(1, 640),
        p["mW0"][:_TRI], p["mW0"][_TRI:], p["mb0"].reshape(1, 512),
        p["mg0"].reshape(1, 512), p["mbb0"].reshape(1, 512),
        p["mW1"], p["mb1"].reshape(1, 256),
        p["mg1"].reshape(1, 256), p["mbb1"].reshape(1, 256),
        p["mW2"], p["mb2"].reshape(1, 256),
        p["mg2"].reshape(1, 256), p["mbb2"].reshape(1, 256),
        p["mW3"], p["mb3"].reshape(1, 10),
    ]
    return pl.pallas_call(
        _head_body,
        out_shape=jax.ShapeDtypeStruct((G, 10), jnp.float32),
    )(*args)


_IU, _JU = np.triu_indices(A, k=1)
_TIDX = np.asarray(_IU * A + _JU + 8192, dtype=np.int32)


def kernel(x, edge_vectors, edge_attr, edge_index, batch, params):
    del batch
    src = edge_index[0].astype(jnp.int32)
    dst = edge_index[1].astype(jnp.int32)
    ea_t = edge_attr.reshape(1, E)

    zt, x0, prt, px = _front_stage(edge_vectors, ea_t, x,
                                   params["W0"], params["l0"]["Wproj"])

    tidxf = jnp.asarray(_TIDX).view(jnp.float32)
    pools = [px.reshape(G, 64)]
    hs = []
    nf = x0
    xt = None
    for i in range(3):
        p = params["l%d" % i]
        if i == 0:
            aft, xt = _sc_scatter_triu(prt, zt, src, dst,
                                       x.reshape(G, A * A), tidxf)
            xt = xt.reshape(G, _TRI)
        else:
            aft = _sc_scatter(prt, zt, src, dst)
        aft = aft.reshape(C * L, N)
        wpn = params["l%d" % (i + 1)]["Wproj"] if i < 2 else None
        if wpn is not None:
            h, prt, ph = _dense_stage(aft, nf, x, p, wpn)
        else:
            h, ph = _dense_stage(aft, nf, x, p, None)
        pools.append(ph.reshape(G, p["W1"].shape[1]))
        hs.append(h)
        nf = h

    pooled = jnp.concatenate(pools, axis=1)
    return _head(xt, pooled, params)
